# baseline (speedup 1.0000x reference)
import functools

import jax
import jax.numpy as jnp
from jax.experimental import pallas as pl
from jax.experimental.pallas import tpu as pltpu


def _se_kernel(x_ref, ones_ref, w1_ref, b1_ref, w2_ref, b2_ref, o_ref,
               *, inv_hw):
    x = x_ref[...]
    pooled = jax.lax.dot(x, ones_ref[...],
                         preferred_element_type=jnp.float32) * inv_hw
    h = jnp.dot(w1_ref[...], pooled, preferred_element_type=jnp.float32)
    h = jnp.maximum(h + b1_ref[...], 0.0)
    g = jnp.dot(w2_ref[...], h, preferred_element_type=jnp.float32)
    g = jax.nn.sigmoid(g + b2_ref[...])
    o_ref[...] = x * g


def kernel(x, w1, b1, w2, b2):
    B, C, H, W = x.shape
    Ch = w1.shape[0]
    HW = H * W
    itemsize = jnp.dtype(x.dtype).itemsize

    x3 = x.reshape(B, C, HW)
    ones = jnp.ones((HW, 1), jnp.float32)
    w1f = jnp.asarray(w1, jnp.float32)
    b1c = jnp.asarray(b1, jnp.float32).reshape(Ch, 1)
    w2f = jnp.asarray(w2, jnp.float32)
    b2c = jnp.asarray(b2, jnp.float32).reshape(C, 1)

    slab_bytes = C * HW * itemsize
    vmem_limit = int(min(96 << 20, 4 * slab_bytes + (8 << 20)))

    out3 = pl.pallas_call(
        functools.partial(_se_kernel, inv_hw=1.0 / HW),
        out_shape=jax.ShapeDtypeStruct((B, C, HW), x.dtype),
        grid=(B,),
        in_specs=[pl.BlockSpec((None, C, HW), lambda b: (b, 0, 0)),
                  pl.BlockSpec((HW, 1), lambda b: (0, 0)),
                  pl.BlockSpec((Ch, C), lambda b: (0, 0)),
                  pl.BlockSpec((Ch, 1), lambda b: (0, 0)),
                  pl.BlockSpec((C, Ch), lambda b: (0, 0)),
                  pl.BlockSpec((C, 1), lambda b: (0, 0))],
        out_specs=pl.BlockSpec((None, C, HW), lambda b: (b, 0, 0)),
        compiler_params=pltpu.CompilerParams(
            dimension_semantics=("parallel",),
            vmem_limit_bytes=vmem_limit),
        cost_estimate=pl.CostEstimate(
            flops=2 * B * HW * C + 2 * B * HW * C + 4 * B * C * Ch,
            transcendentals=B * C,
            bytes_accessed=2 * B * HW * C * itemsize),
    )(x3, ones, w1f, b1c, w2f, b2c)
    return out3.reshape(B, C, H, W)
